# Initial kernel scaffold; baseline (speedup 1.0000x reference)
import jax
import jax.numpy as jnp
from jax.experimental import pallas as pl


def kernel(x, edge_index, edge_weight, W1, b1, W2, b2):
    raise NotImplementedError("write your pallas kernel here")


# speedup vs baseline: 3.6543x; 3.6543x over previous
import functools

import jax
import jax.numpy as jnp
from jax import lax
from jax.experimental import pallas as pl
from jax.experimental.pallas import tpu as pltpu
from jax.experimental.pallas import tpu_sc as plsc

N_NODES = 10000
D = 128

NC = 2
NS = 16
NW = NC * NS
L = 16

CH = 128
N_PAD = 10240
ROWS_PER_SUB = N_PAD // NS


def _pad_edges(n_edges):
    per_worker = -(-n_edges // (NW * CH)) * CH
    return per_worker * NW, per_worker


_ROW_BLK = 1000


def _mm_body(x_ref, w_ref, o_ref):
    o_ref[...] = lax.dot_general(
        x_ref[...], w_ref[...], (((1,), (1,)), ((), ())),
        preferred_element_type=jnp.float32)


def _tc_matmul(x, w):
    n = x.shape[0]
    return pl.pallas_call(
        _mm_body,
        grid=(n // _ROW_BLK,),
        in_specs=[
            pl.BlockSpec((_ROW_BLK, D), lambda i: (i, 0)),
            pl.BlockSpec((D, D), lambda i: (0, 0)),
        ],
        out_specs=pl.BlockSpec((_ROW_BLK, D), lambda i: (i, 0)),
        out_shape=jax.ShapeDtypeStruct((n, D), jnp.float32),
    )(x, w)


def _fuse_relu_mm_body(p_ref, b_ref, w_ref, o_ref):
    h = jnp.maximum(p_ref[0] + p_ref[1] + b_ref[...][None, :], 0.0)
    o_ref[...] = lax.dot_general(
        h, w_ref[...], (((1,), (1,)), ((), ())),
        preferred_element_type=jnp.float32)


def _tc_combine_relu_matmul(parts, b, w):
    n = N_NODES
    return pl.pallas_call(
        _fuse_relu_mm_body,
        grid=(n // _ROW_BLK,),
        in_specs=[
            pl.BlockSpec((NC, _ROW_BLK, D), lambda i: (0, i, 0)),
            pl.BlockSpec((D,), lambda i: (0,)),
            pl.BlockSpec((D, D), lambda i: (0, 0)),
        ],
        out_specs=pl.BlockSpec((_ROW_BLK, D), lambda i: (i, 0)),
        out_shape=jax.ShapeDtypeStruct((n, D), jnp.float32),
    )(parts, b, w)


def _combine_bias_body(p_ref, b_ref, o_ref):
    o_ref[...] = p_ref[0] + p_ref[1] + b_ref[...][None, :]


def _tc_combine_bias(parts, b):
    n = N_NODES
    return pl.pallas_call(
        _combine_bias_body,
        grid=(n // _ROW_BLK,),
        in_specs=[
            pl.BlockSpec((NC, _ROW_BLK, D), lambda i: (0, i, 0)),
            pl.BlockSpec((D,), lambda i: (0,)),
        ],
        out_specs=pl.BlockSpec((_ROW_BLK, D), lambda i: (i, 0)),
        out_shape=jax.ShapeDtypeStruct((n, D), jnp.float32),
    )(parts, b)


def _make_sc_edge_kernel(n_chunks_per_worker):
    per_worker = n_chunks_per_worker * CH
    mesh = plsc.VectorSubcoreMesh(
        core_axis_name="c", subcore_axis_name="s", num_cores=NC, num_subcores=NS)

    @functools.partial(
        pl.kernel,
        out_type=jax.ShapeDtypeStruct((NC, N_PAD, D), jnp.float32),
        mesh=mesh,
        scratch_types=[
            pltpu.VMEM((CH,), jnp.int32),
            pltpu.VMEM((CH,), jnp.int32),
            pltpu.VMEM((CH,), jnp.float32),
            pltpu.VMEM((CH, D), jnp.float32),
            pltpu.VMEM_SHARED((N_PAD, D), jnp.float32),
            pltpu.SemaphoreType.DMA,
        ],
    )
    def sc_edge(m_hbm, src_hbm, dst_hbm, w_hbm, zeros_hbm, out_hbm,
                sidx, didx, wv, rows, acc, sem):
        cid = lax.axis_index("c")
        sid = lax.axis_index("s")
        wid = cid * NS + sid

        pltpu.sync_copy(zeros_hbm, acc.at[pl.ds(sid * ROWS_PER_SUB, ROWS_PER_SUB)])
        plsc.subcore_barrier()

        base = wid * per_worker

        def chunk_body(o, _):
            off = base + o * CH
            pltpu.sync_copy(src_hbm.at[pl.ds(off, CH)], sidx)
            pltpu.sync_copy(w_hbm.at[pl.ds(off, CH)], wv)
            pltpu.sync_copy(dst_hbm.at[pl.ds(off, CH)], didx)
            pltpu.async_copy(m_hbm.at[sidx], rows, sem).wait()

            def scale_body(g, _):
                wvec = wv[pl.ds(g * L, L)]
                for t in range(L):
                    s = wvec[t]
                    j = g * L + t
                    for c in range(D // L):
                        sl = pl.ds(c * L, L)
                        rows[j, sl] = rows[j, sl] * s
                return 0

            lax.fori_loop(0, CH // L, scale_body, 0)

            pltpu.sync_copy(rows, acc.at[didx], add=True)
            return 0

        lax.fori_loop(0, n_chunks_per_worker, chunk_body, 0)
        plsc.subcore_barrier()

        pltpu.sync_copy(
            acc.at[pl.ds(sid * ROWS_PER_SUB, ROWS_PER_SUB)],
            out_hbm.at[cid, pl.ds(sid * ROWS_PER_SUB, ROWS_PER_SUB)])

    return sc_edge


def kernel(x, edge_index, edge_weight, W1, b1, W2, b2):
    n_edges = edge_index.shape[1]
    e_pad, per_worker = _pad_edges(n_edges)
    pad = e_pad - n_edges

    src = edge_index[0]
    dst = edge_index[1]
    w = edge_weight
    if pad:
        src = jnp.concatenate([src, jnp.zeros((pad,), jnp.int32)])
        dst = jnp.concatenate([dst, jnp.zeros((pad,), jnp.int32)])
        w = jnp.concatenate([w, jnp.zeros((pad,), jnp.float32)])

    zeros = jnp.zeros((ROWS_PER_SUB, D), jnp.float32)
    sc_edge = _make_sc_edge_kernel(per_worker // CH)

    m1 = _tc_matmul(x, W1)
    parts1 = sc_edge(m1, src, dst, w, zeros)
    m2 = _tc_combine_relu_matmul(parts1, b1, W2)
    parts2 = sc_edge(m2, src, dst, w, zeros)
    return _tc_combine_bias(parts2, b2)
